# Initial kernel scaffold; baseline (speedup 1.0000x reference)
import functools

import jax
import jax.numpy as jnp
from jax import lax
from jax.experimental import pallas as pl
from jax.experimental.pallas import tpu as pltpu
from jax.experimental.pallas import tpu_sc as plsc

N = 10000
E = 160000
F = 128
DE = 4
H = 20
C1 = 16
C2 = 32
G = 64

NC = 2
NS = 16
NW = NC * NS
CH = 128
NPAD = 10016
EPAD = 163840
EPW = EPAD // NW
NCHUNK = EPW // CH
RPT = NPAD // NS
TGW = 336
BM = 2504

_mesh = plsc.VectorSubcoreMesh(core_axis_name="c", subcore_axis_name="s")


def _mm_body(x_ref, w_ref, o_ref):
    o_ref[...] = x_ref[...] @ w_ref[...]


def _emlp_body(e_ref, w0_ref, b0_ref, w1_ref, b1_ref, o_ref):
    h1 = jnp.maximum(e_ref[...] @ w0_ref[...] + b0_ref[...], 0.0)
    o_ref[...] = jnp.maximum(h1 @ w1_ref[...] + b1_ref[...], 0.0)


def _node1_body(a0_ref, a1_ref, x_ref, wr_ref, br_ref, y1_ref, n32_ref):
    a0 = a0_ref[...]
    a1 = a1_ref[...]
    agg = a0[:, :C1] + a1[:, :C1]
    deg = a0[:, C1:C1 + 1] + a1[:, C1:C1 + 1] + 1.0
    x1 = jnp.maximum(agg + x_ref[...] @ wr_ref[...] + br_ref[...], 0.0)
    nrm = lax.rsqrt(deg)
    y1_ref[...] = x1 * nrm
    n32_ref[...] = jnp.broadcast_to(nrm, n32_ref.shape)


def _node2_body(sa_ref, sb_ref, y1_ref, n32_ref, wg_ref, bg_ref, y2_ref):
    n32 = n32_ref[...]
    p = n32[:, :C1] * (sa_ref[...] + sb_ref[...] + y1_ref[...])
    x2 = jnp.maximum(p @ wg_ref[...] + bg_ref[...], 0.0)
    y2_ref[...] = x2 * n32


def _final_body(sa_ref, sb_ref, y2_ref, n32_ref, irow_ref, wg_ref, bg_ref,
                wd1_ref, bd1_ref, wd2_ref, bd2_ref, o_ref, pool_ref):
    t = pl.program_id(0)
    p = n32_ref[...] * (sa_ref[...] + sb_ref[...] + y2_ref[...])
    x3 = jnp.maximum(p @ wg_ref[...] + bg_ref[...], 0.0)
    rid = t * BM + lax.broadcasted_iota(jnp.int32, (BM, 1), 0)
    x3 = jnp.where(rid < N, x3, 0.0)
    gid = irow_ref[...]
    oh = (lax.broadcasted_iota(jnp.float32, (G, BM), 0) == gid)
    contrib = oh.astype(jnp.float32) @ x3

    @pl.when(t == 0)
    def _():
        pool_ref[...] = jnp.zeros_like(pool_ref)

    pool_ref[...] += contrib

    @pl.when(t == pl.num_programs(0) - 1)
    def _():
        g = pool_ref[...]
        o_ref[...] = (g @ wd1_ref[...] + bd1_ref[...]) @ wd2_ref[...] + bd2_ref[...]


def _zero_shared(zbuf, acc, sid, width):
    nv = width // 16

    def _z(j, _):
        for v in range(nv):
            zbuf[j, pl.ds(v * 16, 16)] = jnp.zeros((16,), jnp.float32)
        return 0

    lax.fori_loop(0, RPT, _z, 0)
    pltpu.sync_copy(zbuf, acc.at[pl.ds(sid * RPT, RPT)])


def _ecc_body(tg_hbm, h_hbm, src_hbm, dst_hbm, out_hbm,
              idx_s, idx_d, rows, hbuf, msg, zbuf, acc, sem):
    cid = lax.axis_index("c")
    sid = lax.axis_index("s")
    wid = sid * NC + cid

    _zero_shared(zbuf, acc, sid, 2 * C1)
    plsc.subcore_barrier()

    cvec = jnp.where(lax.iota(jnp.int32, 16) == 0, 1.0, 0.0)

    def _c(j, _):
        msg[j, pl.ds(C1, 16)] = cvec
        return 0

    lax.fori_loop(0, CH, _c, 0)

    ebase = wid * EPW

    def _chunk(c, _):
        base = pl.multiple_of(ebase + c * CH, CH)
        pltpu.sync_copy(src_hbm.at[pl.ds(base, CH)], idx_s)
        pltpu.sync_copy(dst_hbm.at[pl.ds(base, CH)], idx_d)
        pltpu.sync_copy(h_hbm.at[pl.ds(base, CH)], hbuf)
        pltpu.async_copy(tg_hbm.at[idx_s], rows, sem).wait()

        def _edge(k, _):
            m = rows[k, pl.ds(H * C1, 16)]
            for hh in range(H):
                m = m + rows[k, pl.ds(hh * 16, 16)] * hbuf[k, hh]
            msg[k, pl.ds(0, 16)] = m
            return 0

        lax.fori_loop(0, CH, _edge, 0)
        pltpu.sync_copy(msg, acc.at[idx_d], add=True)
        return 0

    lax.fori_loop(0, NCHUNK, _chunk, 0)

    plsc.subcore_barrier()
    pltpu.sync_copy(acc.at[pl.ds(sid * RPT, RPT)],
                    out_hbm.at[cid, pl.ds(sid * RPT, RPT)])


def _make_seg_body(width):
    def _seg_body(y_hbm, src_hbm, dst_hbm, out_hbm,
                  idx_s, idx_d, rows, zbuf, acc, sem):
        cid = lax.axis_index("c")
        sid = lax.axis_index("s")
        wid = sid * NC + cid

        _zero_shared(zbuf, acc, sid, width)
        plsc.subcore_barrier()

        ebase = wid * EPW

        def _chunk(c, _):
            base = pl.multiple_of(ebase + c * CH, CH)
            pltpu.sync_copy(src_hbm.at[pl.ds(base, CH)], idx_s)
            pltpu.sync_copy(dst_hbm.at[pl.ds(base, CH)], idx_d)
            pltpu.async_copy(y_hbm.at[idx_s], rows, sem).wait()
            pltpu.sync_copy(rows, acc.at[idx_d], add=True)
            return 0

        lax.fori_loop(0, NCHUNK, _chunk, 0)

        plsc.subcore_barrier()
        pltpu.sync_copy(acc.at[pl.ds(sid * RPT, RPT)],
                        out_hbm.at[cid, pl.ds(sid * RPT, RPT)])

    return _seg_body


_ecc_call = functools.partial(
    pl.kernel,
    _ecc_body,
    out_type=jax.ShapeDtypeStruct((NC, NPAD, 2 * C1), jnp.float32),
    mesh=_mesh,
    scratch_types=[
        pltpu.VMEM((CH,), jnp.int32),
        pltpu.VMEM((CH,), jnp.int32),
        pltpu.VMEM((CH, TGW), jnp.float32),
        pltpu.VMEM((CH, 2 * C1), jnp.float32),
        pltpu.VMEM((CH, 2 * C1), jnp.float32),
        pltpu.VMEM((RPT, 2 * C1), jnp.float32),
        pltpu.VMEM_SHARED((NPAD, 2 * C1), jnp.float32),
        pltpu.SemaphoreType.DMA,
    ],
)


def _seg_call(width):
    return functools.partial(
        pl.kernel,
        _make_seg_body(width),
        out_type=jax.ShapeDtypeStruct((NC, NPAD, width), jnp.float32),
        mesh=_mesh,
        scratch_types=[
            pltpu.VMEM((CH,), jnp.int32),
            pltpu.VMEM((CH,), jnp.int32),
            pltpu.VMEM((CH, width), jnp.float32),
            pltpu.VMEM((RPT, width), jnp.float32),
            pltpu.VMEM_SHARED((NPAD, width), jnp.float32),
            pltpu.SemaphoreType.DMA,
        ],
    )


def kernel(x, edge_index, e, i, W0, b0, W1, b1, W2, b2, Wr, br,
           Wg1, bg1, Wg2, bg2, Wd1, bd1, Wd2, bd2):
    f32 = jnp.float32
    src = edge_index[0].astype(jnp.int32)
    dst = edge_index[1].astype(jnp.int32)
    pad_idx = jnp.full((EPAD - E,), N, jnp.int32)
    src_p = jnp.concatenate([src, pad_idx])
    dst_p = jnp.concatenate([dst, pad_idx])
    e_p = jnp.pad(e, ((0, EPAD - E), (0, 0)))

    W2r = W2.reshape(H, F, C1)
    wcat = jnp.concatenate(
        [jnp.transpose(W2r, (1, 0, 2)).reshape(F, H * C1), b2.reshape(F, C1)],
        axis=1)
    W1p = jnp.pad(W1, ((0, 0), (0, C2 - H)))
    b1p = jnp.pad(b1, (0, C2 - H)).reshape(1, C2)
    b0r = b0.reshape(1, H)
    brr = br.reshape(1, C1)
    bg1r = bg1.reshape(1, C2)
    bg2r = bg2.reshape(1, C2)
    bd1r = bd1.reshape(1, 16)
    bd2r = bd2.reshape(1, 1)
    i_p = jnp.pad(i.astype(f32).reshape(1, N), ((0, 0), (0, NPAD - N)),
                  constant_values=-1.0)

    tg = pl.pallas_call(
        _mm_body,
        grid=(NPAD // BM,),
        in_specs=[pl.BlockSpec((BM, F), lambda t: (t, 0)),
                  pl.BlockSpec((F, TGW), lambda t: (0, 0))],
        out_specs=pl.BlockSpec((BM, TGW), lambda t: (t, 0)),
        out_shape=jax.ShapeDtypeStruct((NPAD, TGW), f32),
    )(x, wcat)

    BE = 2048
    h_e = pl.pallas_call(
        _emlp_body,
        grid=(EPAD // BE,),
        in_specs=[pl.BlockSpec((BE, DE), lambda t: (t, 0)),
                  pl.BlockSpec((DE, H), lambda t: (0, 0)),
                  pl.BlockSpec((1, H), lambda t: (0, 0)),
                  pl.BlockSpec((H, C2), lambda t: (0, 0)),
                  pl.BlockSpec((1, C2), lambda t: (0, 0))],
        out_specs=pl.BlockSpec((BE, C2), lambda t: (t, 0)),
        out_shape=jax.ShapeDtypeStruct((EPAD, C2), f32),
    )(e_p, W0, b0r, W1p, b1p)

    agg2 = _ecc_call()(tg, h_e, src_p, dst_p)

    y1, n32 = pl.pallas_call(
        _node1_body,
        grid=(NPAD // BM,),
        in_specs=[pl.BlockSpec((BM, 2 * C1), lambda t: (t, 0)),
                  pl.BlockSpec((BM, 2 * C1), lambda t: (t, 0)),
                  pl.BlockSpec((BM, F), lambda t: (t, 0)),
                  pl.BlockSpec((F, C1), lambda t: (0, 0)),
                  pl.BlockSpec((1, C1), lambda t: (0, 0))],
        out_specs=[pl.BlockSpec((BM, C1), lambda t: (t, 0)),
                   pl.BlockSpec((BM, C2), lambda t: (t, 0))],
        out_shape=[jax.ShapeDtypeStruct((NPAD, C1), f32),
                   jax.ShapeDtypeStruct((NPAD, C2), f32)],
    )(agg2[0], agg2[1], x, Wr, brr)

    s1 = _seg_call(C1)()(y1, src_p, dst_p)

    y2 = pl.pallas_call(
        _node2_body,
        grid=(NPAD // BM,),
        in_specs=[pl.BlockSpec((BM, C1), lambda t: (t, 0)),
                  pl.BlockSpec((BM, C1), lambda t: (t, 0)),
                  pl.BlockSpec((BM, C1), lambda t: (t, 0)),
                  pl.BlockSpec((BM, C2), lambda t: (t, 0)),
                  pl.BlockSpec((C1, C2), lambda t: (0, 0)),
                  pl.BlockSpec((1, C2), lambda t: (0, 0))],
        out_specs=pl.BlockSpec((BM, C2), lambda t: (t, 0)),
        out_shape=jax.ShapeDtypeStruct((NPAD, C2), f32),
    )(s1[0], s1[1], y1, n32, Wg1, bg1r)

    s2 = _seg_call(C2)()(y2, src_p, dst_p)

    out = pl.pallas_call(
        _final_body,
        grid=(NPAD // BM,),
        in_specs=[pl.BlockSpec((BM, C2), lambda t: (t, 0)),
                  pl.BlockSpec((BM, C2), lambda t: (t, 0)),
                  pl.BlockSpec((BM, C2), lambda t: (t, 0)),
                  pl.BlockSpec((BM, C2), lambda t: (t, 0)),
                  pl.BlockSpec((1, BM), lambda t: (0, t)),
                  pl.BlockSpec((C2, C2), lambda t: (0, 0)),
                  pl.BlockSpec((1, C2), lambda t: (0, 0)),
                  pl.BlockSpec((C2, 16), lambda t: (0, 0)),
                  pl.BlockSpec((1, 16), lambda t: (0, 0)),
                  pl.BlockSpec((16, 1), lambda t: (0, 0)),
                  pl.BlockSpec((1, 1), lambda t: (0, 0))],
        out_specs=pl.BlockSpec((G, 1), lambda t: (0, 0)),
        out_shape=jax.ShapeDtypeStruct((G, 1), f32),
        scratch_shapes=[pltpu.VMEM((G, C2), f32)],
    )(s2[0], s2[1], y2, n32, i_p, Wg2, bg2r, Wd1, bd1r, Wd2, bd2r)

    return out


# speedup vs baseline: 2.9191x; 2.9191x over previous
import functools

import jax
import jax.numpy as jnp
from jax import lax
from jax.experimental import pallas as pl
from jax.experimental.pallas import tpu as pltpu
from jax.experimental.pallas import tpu_sc as plsc

N = 10000
E = 160000
F = 128
DE = 4
H = 20
C1 = 16
C2 = 32
G = 64

NC = 2
NS = 16
NW = NC * NS
CH = 128
NPAD = 10112
EPAD = 163840
EPW = EPAD // NW
NCHUNK = EPW // CH
ECH = 64
ECHUNK = EPW // ECH
HROWS = ECH // 4
RPT = NPAD // NS
TGW = 384
BM = 2528

_mesh = plsc.VectorSubcoreMesh(core_axis_name="c", subcore_axis_name="s")


def _mm_body(x_ref, w_ref, o_ref):
    o_ref[...] = x_ref[...] @ w_ref[...]


def _emlp_body(e_ref, w0_ref, b0_ref, w1_ref, b1_ref, o_ref):
    h1 = jnp.maximum(e_ref[...] @ w0_ref[...] + b0_ref[...], 0.0)
    o_ref[...] = jnp.maximum(h1 @ w1_ref[...] + b1_ref[...], 0.0)


def _node1_body(a0_ref, a1_ref, x_ref, wr_ref, br_ref, y1_ref, n32_ref):
    a0 = a0_ref[...]
    a1 = a1_ref[...]
    agg = a0[:, :C1] + a1[:, :C1]
    deg = a0[:, C1:C1 + 1] + a1[:, C1:C1 + 1] + 1.0
    x1 = jnp.maximum(agg + x_ref[...] @ wr_ref[...] + br_ref[...], 0.0)
    nrm = lax.rsqrt(deg)
    y1 = x1 * nrm
    y1_ref[...] = jnp.concatenate([y1] * (F // C1), axis=1)
    n32_ref[...] = jnp.broadcast_to(nrm, n32_ref.shape)


def _node2_body(sa_ref, sb_ref, y1_ref, n32_ref, wg_ref, bg_ref, y2_ref):
    n32 = n32_ref[...]
    p = n32[:, :C1] * (sa_ref[:, :C1] + sb_ref[:, :C1] + y1_ref[:, :C1])
    x2 = jnp.maximum(p @ wg_ref[...] + bg_ref[...], 0.0)
    y2_ref[...] = jnp.concatenate([x2 * n32] * (F // C2), axis=1)


def _final_body(sa_ref, sb_ref, y2_ref, n32_ref, irow_ref, wg_ref, bg_ref,
                wd1_ref, bd1_ref, wd2_ref, bd2_ref, o_ref, pool_ref):
    t = pl.program_id(0)
    p = n32_ref[...] * (sa_ref[:, :C2] + sb_ref[:, :C2] + y2_ref[:, :C2])
    x3 = jnp.maximum(p @ wg_ref[...] + bg_ref[...], 0.0)
    rid = t * BM + lax.broadcasted_iota(jnp.int32, (BM, 1), 0)
    x3 = jnp.where(rid < N, x3, 0.0)
    gid = irow_ref[0]
    oh = (lax.broadcasted_iota(jnp.int32, (G, BM), 0).astype(jnp.float32)
          == gid)
    contrib = oh.astype(jnp.float32) @ x3

    @pl.when(t == 0)
    def _():
        pool_ref[...] = jnp.zeros_like(pool_ref)

    pool_ref[...] += contrib

    @pl.when(t == pl.num_programs(0) - 1)
    def _():
        g = pool_ref[...]
        o_ref[...] = (g @ wd1_ref[...] + bd1_ref[...]) @ wd2_ref[...] + bd2_ref[...]


def _zero_shared(zbuf, acc, sid, width):
    nv = width // 16

    def _z(j, _):
        for v in range(nv):
            zbuf[j, pl.ds(v * 16, 16)] = jnp.zeros((16,), jnp.float32)
        return 0

    lax.fori_loop(0, 8, _z, 0)

    def _cp(q, _):
        off = pl.multiple_of(sid * RPT + q * 8, 8)
        pltpu.sync_copy(zbuf, acc.at[pl.ds(off, 8)])
        return 0

    lax.fori_loop(0, RPT // 8, _cp, 0)


def _ecc_body(tg_hbm, h_hbm, src_hbm, dst_hbm, out_hbm,
              idx_s, idx_d, rows, hbuf, msg, zbuf, acc, sem):
    cid = lax.axis_index("c")
    sid = lax.axis_index("s")
    wid = sid * NC + cid

    _zero_shared(zbuf, acc, sid, F)
    plsc.subcore_barrier()

    cvec = jnp.where(lax.iota(jnp.int32, 16) == 0, 1.0, 0.0)
    zv = jnp.zeros((16,), jnp.float32)

    def _c(j, _):
        msg[j, pl.ds(C1, 16)] = cvec
        for v in range(2, F // 16):
            msg[j, pl.ds(v * 16, 16)] = zv
        return 0

    lax.fori_loop(0, ECH, _c, 0)

    ebase = wid * EPW
    hbase0 = wid * (EPW // 4)

    def _chunk(c, _):
        base = pl.multiple_of(ebase + c * ECH, ECH)
        hbase = pl.multiple_of(hbase0 + c * HROWS, 8)
        pltpu.sync_copy(src_hbm.at[pl.ds(base, ECH)], idx_s)
        pltpu.sync_copy(dst_hbm.at[pl.ds(base, ECH)], idx_d)
        pltpu.sync_copy(h_hbm.at[pl.ds(hbase, HROWS)], hbuf)
        pltpu.async_copy(tg_hbm.at[idx_s], rows, sem).wait()

        def _edge(k, _):
            r = k // 4
            off = (k - 4 * r) * 32
            hv0 = hbuf[r, pl.ds(off, 16)]
            hv1 = hbuf[r, pl.ds(off + 16, 16)]
            m = rows[k, pl.ds(H * C1, 16)]
            for hh in range(H):
                s = hv0[hh] if hh < 16 else hv1[hh - 16]
                m = m + rows[k, pl.ds(hh * 16, 16)] * s
            msg[k, pl.ds(0, 16)] = m
            return 0

        lax.fori_loop(0, ECH, _edge, 0)
        pltpu.sync_copy(msg, acc.at[idx_d], add=True)
        return 0

    lax.fori_loop(0, ECHUNK, _chunk, 0)

    plsc.subcore_barrier()
    pltpu.sync_copy(acc.at[pl.ds(sid * RPT, RPT)],
                    out_hbm.at[cid, pl.ds(sid * RPT, RPT)])


def _make_seg_body(width):
    def _seg_body(y_hbm, src_hbm, dst_hbm, out_hbm,
                  idx_s, idx_d, rows, zbuf, acc, sem):
        cid = lax.axis_index("c")
        sid = lax.axis_index("s")
        wid = sid * NC + cid

        _zero_shared(zbuf, acc, sid, width)
        plsc.subcore_barrier()

        ebase = wid * EPW

        def _chunk(c, _):
            base = pl.multiple_of(ebase + c * CH, CH)
            pltpu.sync_copy(src_hbm.at[pl.ds(base, CH)], idx_s)
            pltpu.sync_copy(dst_hbm.at[pl.ds(base, CH)], idx_d)
            pltpu.async_copy(y_hbm.at[idx_s], rows, sem).wait()
            pltpu.sync_copy(rows, acc.at[idx_d], add=True)
            return 0

        lax.fori_loop(0, NCHUNK, _chunk, 0)

        plsc.subcore_barrier()
        pltpu.sync_copy(acc.at[pl.ds(sid * RPT, RPT)],
                        out_hbm.at[cid, pl.ds(sid * RPT, RPT)])

    return _seg_body


_ecc_call = functools.partial(
    pl.kernel,
    _ecc_body,
    out_type=jax.ShapeDtypeStruct((NC, NPAD, F), jnp.float32),
    mesh=_mesh,
    scratch_types=[
        pltpu.VMEM((ECH,), jnp.int32),
        pltpu.VMEM((ECH,), jnp.int32),
        pltpu.VMEM((ECH, TGW), jnp.float32),
        pltpu.VMEM((HROWS, F), jnp.float32),
        pltpu.VMEM((ECH, F), jnp.float32),
        pltpu.VMEM((8, F), jnp.float32),
        pltpu.VMEM_SHARED((NPAD, F), jnp.float32),
        pltpu.SemaphoreType.DMA,
    ],
)


_seg_call = functools.partial(
    pl.kernel,
    _make_seg_body(F),
    out_type=jax.ShapeDtypeStruct((NC, NPAD, F), jnp.float32),
    mesh=_mesh,
    scratch_types=[
        pltpu.VMEM((CH,), jnp.int32),
        pltpu.VMEM((CH,), jnp.int32),
        pltpu.VMEM((CH, F), jnp.float32),
        pltpu.VMEM((8, F), jnp.float32),
        pltpu.VMEM_SHARED((NPAD, F), jnp.float32),
        pltpu.SemaphoreType.DMA,
    ],
)


def kernel(x, edge_index, e, i, W0, b0, W1, b1, W2, b2, Wr, br,
           Wg1, bg1, Wg2, bg2, Wd1, bd1, Wd2, bd2):
    f32 = jnp.float32
    src = edge_index[0].astype(jnp.int32)
    dst = edge_index[1].astype(jnp.int32)
    pad_idx = jnp.full((EPAD - E,), N, jnp.int32)
    src_p = jnp.concatenate([src, pad_idx])
    dst_p = jnp.concatenate([dst, pad_idx])
    e_p = jnp.pad(e, ((0, EPAD - E), (0, 0)))

    W2r = W2.reshape(H, F, C1)
    wcat = jnp.concatenate(
        [jnp.transpose(W2r, (1, 0, 2)).reshape(F, H * C1), b2.reshape(F, C1),
         jnp.zeros((F, TGW - H * C1 - C1), f32)],
        axis=1)
    W1p = jnp.pad(W1, ((0, 0), (0, C2 - H)))
    b1p = jnp.pad(b1, (0, C2 - H)).reshape(1, C2)
    b0r = b0.reshape(1, H)
    brr = br.reshape(1, C1)
    bg1r = bg1.reshape(1, C2)
    bg2r = bg2.reshape(1, C2)
    bd1r = bd1.reshape(1, 16)
    bd2r = bd2.reshape(1, 1)
    i_p = jnp.pad(i.astype(f32), (0, NPAD - N),
                  constant_values=-1.0).reshape(NPAD // BM, 1, BM)

    tg = pl.pallas_call(
        _mm_body,
        grid=(NPAD // BM,),
        in_specs=[pl.BlockSpec((BM, F), lambda t: (t, 0)),
                  pl.BlockSpec((F, TGW), lambda t: (0, 0))],
        out_specs=pl.BlockSpec((BM, TGW), lambda t: (t, 0)),
        out_shape=jax.ShapeDtypeStruct((NPAD, TGW), f32),
    )(x, wcat)

    BE = 2048
    h_e = pl.pallas_call(
        _emlp_body,
        grid=(EPAD // BE,),
        in_specs=[pl.BlockSpec((BE, DE), lambda t: (t, 0)),
                  pl.BlockSpec((DE, H), lambda t: (0, 0)),
                  pl.BlockSpec((1, H), lambda t: (0, 0)),
                  pl.BlockSpec((H, C2), lambda t: (0, 0)),
                  pl.BlockSpec((1, C2), lambda t: (0, 0))],
        out_specs=pl.BlockSpec((BE, C2), lambda t: (t, 0)),
        out_shape=jax.ShapeDtypeStruct((EPAD, C2), f32),
    )(e_p, W0, b0r, W1p, b1p)

    _USE_SC = {"ecc": True, "seg1": True, "seg2": True}
    if _USE_SC["ecc"]:
        h4 = h_e.reshape(EPAD // 4, F)
        agg2 = _ecc_call()(tg, h4, src_p, dst_p)
    else:
        rowsg = tg[src_p]
        msum = ((rowsg[:, :H * C1].reshape(EPAD, H, C1)
                 * h_e[:, :H, None]).sum(1) + rowsg[:, H * C1:H * C1 + C1])
        wide = jnp.concatenate(
            [msum, jnp.ones((EPAD, 1), f32), jnp.zeros((EPAD, F - C1 - 1), f32)],
            axis=1)
        a0 = jax.ops.segment_sum(wide, dst_p, num_segments=NPAD)
        agg2 = jnp.stack([a0, jnp.zeros_like(a0)])

    y1, n32 = pl.pallas_call(
        _node1_body,
        grid=(NPAD // BM,),
        in_specs=[pl.BlockSpec((BM, F), lambda t: (t, 0)),
                  pl.BlockSpec((BM, F), lambda t: (t, 0)),
                  pl.BlockSpec((BM, F), lambda t: (t, 0)),
                  pl.BlockSpec((F, C1), lambda t: (0, 0)),
                  pl.BlockSpec((1, C1), lambda t: (0, 0))],
        out_specs=[pl.BlockSpec((BM, F), lambda t: (t, 0)),
                   pl.BlockSpec((BM, C2), lambda t: (t, 0))],
        out_shape=[jax.ShapeDtypeStruct((NPAD, F), f32),
                   jax.ShapeDtypeStruct((NPAD, C2), f32)],
    )(agg2[0], agg2[1], x, Wr, brr)

    if _USE_SC["seg1"]:
        s1 = _seg_call()(y1, src_p, dst_p)
    else:
        s1a = jax.ops.segment_sum(y1[src_p], dst_p, num_segments=NPAD)
        s1 = jnp.stack([s1a, jnp.zeros_like(s1a)])

    y2 = pl.pallas_call(
        _node2_body,
        grid=(NPAD // BM,),
        in_specs=[pl.BlockSpec((BM, F), lambda t: (t, 0)),
                  pl.BlockSpec((BM, F), lambda t: (t, 0)),
                  pl.BlockSpec((BM, F), lambda t: (t, 0)),
                  pl.BlockSpec((BM, C2), lambda t: (t, 0)),
                  pl.BlockSpec((C1, C2), lambda t: (0, 0)),
                  pl.BlockSpec((1, C2), lambda t: (0, 0))],
        out_specs=pl.BlockSpec((BM, F), lambda t: (t, 0)),
        out_shape=jax.ShapeDtypeStruct((NPAD, F), f32),
    )(s1[0], s1[1], y1, n32, Wg1, bg1r)

    if _USE_SC["seg2"]:
        s2 = _seg_call()(y2, src_p, dst_p)
    else:
        s2a = jax.ops.segment_sum(y2[src_p], dst_p, num_segments=NPAD)
        s2 = jnp.stack([s2a, jnp.zeros_like(s2a)])

    out = pl.pallas_call(
        _final_body,
        grid=(NPAD // BM,),
        in_specs=[pl.BlockSpec((BM, F), lambda t: (t, 0)),
                  pl.BlockSpec((BM, F), lambda t: (t, 0)),
                  pl.BlockSpec((BM, F), lambda t: (t, 0)),
                  pl.BlockSpec((BM, C2), lambda t: (t, 0)),
                  pl.BlockSpec((1, 1, BM), lambda t: (t, 0, 0)),
                  pl.BlockSpec((C2, C2), lambda t: (0, 0)),
                  pl.BlockSpec((1, C2), lambda t: (0, 0)),
                  pl.BlockSpec((C2, 16), lambda t: (0, 0)),
                  pl.BlockSpec((1, 16), lambda t: (0, 0)),
                  pl.BlockSpec((16, 1), lambda t: (0, 0)),
                  pl.BlockSpec((1, 1), lambda t: (0, 0))],
        out_specs=pl.BlockSpec((G, 1), lambda t: (0, 0)),
        out_shape=jax.ShapeDtypeStruct((G, 1), f32),
        scratch_shapes=[pltpu.VMEM((G, C2), f32)],
    )(s2[0], s2[1], y2, n32, i_p, Wg2, bg2r, Wd1, bd1r, Wd2, bd2r)

    return out


# speedup vs baseline: 3.0527x; 1.0457x over previous
import functools

import jax
import jax.numpy as jnp
from jax import lax
from jax.experimental import pallas as pl
from jax.experimental.pallas import tpu as pltpu
from jax.experimental.pallas import tpu_sc as plsc

N = 10000
E = 160000
F = 128
DE = 4
H = 20
C1 = 16
C2 = 32
G = 64

NC = 2
NS = 16
NW = NC * NS
CH = 128
NPAD = 10112
EPAD = 163840
EPW = EPAD // NW
NCHUNK = EPW // CH
ECH = 64
ECHUNK = EPW // ECH
NSLOT = 3
NTRI = ECHUNK // NSLOT
HROWS = ECH // 4
RPT = NPAD // NS
TGW = 384
BM = 2528

_mesh = plsc.VectorSubcoreMesh(core_axis_name="c", subcore_axis_name="s")


def _mm_body(x_ref, w_ref, o_ref):
    o_ref[...] = x_ref[...] @ w_ref[...]


def _emlp_body(e_ref, w0_ref, b0_ref, w1_ref, b1_ref, o_ref):
    h1 = jnp.maximum(e_ref[...] @ w0_ref[...] + b0_ref[...], 0.0)
    o_ref[...] = jnp.maximum(h1 @ w1_ref[...] + b1_ref[...], 0.0)


def _node1_body(a0_ref, a1_ref, x_ref, wr_ref, br_ref, y1_ref, n32_ref):
    a0 = a0_ref[...]
    a1 = a1_ref[...]
    agg = a0[:, :C1] + a1[:, :C1]
    deg = a0[:, C1:C1 + 1] + a1[:, C1:C1 + 1] + 1.0
    x1 = jnp.maximum(agg + x_ref[...] @ wr_ref[...] + br_ref[...], 0.0)
    nrm = lax.rsqrt(deg)
    y1 = x1 * nrm
    y1_ref[...] = jnp.concatenate([y1] * (F // C1), axis=1)
    n32_ref[...] = jnp.broadcast_to(nrm, n32_ref.shape)


def _node2_body(sa_ref, sb_ref, y1_ref, n32_ref, wg_ref, bg_ref, y2_ref):
    n32 = n32_ref[...]
    p = n32[:, :C1] * (sa_ref[:, :C1] + sb_ref[:, :C1] + y1_ref[:, :C1])
    x2 = jnp.maximum(p @ wg_ref[...] + bg_ref[...], 0.0)
    y2_ref[...] = jnp.concatenate([x2 * n32] * (F // C2), axis=1)


def _final_body(sa_ref, sb_ref, y2_ref, n32_ref, irow_ref, wg_ref, bg_ref,
                wd1_ref, bd1_ref, wd2_ref, bd2_ref, o_ref, pool_ref):
    t = pl.program_id(0)
    p = n32_ref[...] * (sa_ref[:, :C2] + sb_ref[:, :C2] + y2_ref[:, :C2])
    x3 = jnp.maximum(p @ wg_ref[...] + bg_ref[...], 0.0)
    rid = t * BM + lax.broadcasted_iota(jnp.int32, (BM, 1), 0)
    x3 = jnp.where(rid < N, x3, 0.0)
    gid = irow_ref[0]
    oh = (lax.broadcasted_iota(jnp.int32, (G, BM), 0).astype(jnp.float32)
          == gid)
    contrib = oh.astype(jnp.float32) @ x3

    @pl.when(t == 0)
    def _():
        pool_ref[...] = jnp.zeros_like(pool_ref)

    pool_ref[...] += contrib

    @pl.when(t == pl.num_programs(0) - 1)
    def _():
        g = pool_ref[...]
        o_ref[...] = (g @ wd1_ref[...] + bd1_ref[...]) @ wd2_ref[...] + bd2_ref[...]


def _zero_shared(zbuf, acc, sid, width):
    nv = width // 16

    def _z(j, _):
        for v in range(nv):
            zbuf[j, pl.ds(v * 16, 16)] = jnp.zeros((16,), jnp.float32)
        return 0

    lax.fori_loop(0, 8, _z, 0)

    def _cp(q, _):
        off = pl.multiple_of(sid * RPT + q * 8, 8)
        pltpu.sync_copy(zbuf, acc.at[pl.ds(off, 8)])
        return 0

    lax.fori_loop(0, RPT // 8, _cp, 0)


def _ecc_compute(rows, hbuf, msg):
    def _edge(k, _):
        r = k // 4
        off = (k - 4 * r) * 32
        hv0 = hbuf[r, pl.ds(off, 16)]
        hv1 = hbuf[r, pl.ds(off + 16, 16)]
        m = rows[k, pl.ds(H * C1, 16)]
        for hh in range(H):
            s = hv0[hh] if hh < 16 else hv1[hh - 16]
            m = m + rows[k, pl.ds(hh * 16, 16)] * s
        msg[k, pl.ds(0, 16)] = m
        return 0

    lax.fori_loop(0, ECH, _edge, 0)


def _ecc_body(tg_hbm, h_hbm, src_hbm, dst_hbm, out_hbm,
              idx_s, idx_d, rows, hbuf, msg, zbuf, acc, sem):
    cid = lax.axis_index("c")
    sid = lax.axis_index("s")
    wid = sid * NC + cid

    _zero_shared(zbuf, acc, sid, 2 * C1)

    cvec = jnp.where(lax.iota(jnp.int32, 16) == 0, 1.0, 0.0)

    def _c(j, _):
        msg[j, pl.ds(C1, 16)] = cvec
        return 0

    lax.fori_loop(0, ECH, _c, 0)

    ebase = wid * EPW
    hbase0 = wid * (EPW // 4)
    plsc.subcore_barrier()

    def _chunk(c, _):
        base = pl.multiple_of(ebase + c * ECH, ECH)
        hbase = pl.multiple_of(hbase0 + c * HROWS, 8)
        pltpu.sync_copy(src_hbm.at[pl.ds(base, ECH)], idx_s)
        pltpu.sync_copy(dst_hbm.at[pl.ds(base, ECH)], idx_d)
        pltpu.sync_copy(h_hbm.at[pl.ds(hbase, HROWS)], hbuf)
        pltpu.async_copy(tg_hbm.at[idx_s], rows, sem).wait()
        _ecc_compute(rows, hbuf, msg)
        pltpu.sync_copy(msg, acc.at[idx_d], add=True)
        return 0

    lax.fori_loop(0, ECHUNK, _chunk, 0)

    plsc.subcore_barrier()
    pltpu.sync_copy(acc.at[pl.ds(sid * RPT, RPT)],
                    out_hbm.at[cid, pl.ds(sid * RPT, RPT)])


def _seg_body(y_hbm, src_hbm, dst_hbm, out_hbm,
              idxs0, idxs1, idxd0, idxd1, rows0, rows1, zbuf, acc,
              gs0, gs1, ss0, ss1):
    cid = lax.axis_index("c")
    sid = lax.axis_index("s")
    wid = sid * NC + cid

    _zero_shared(zbuf, acc, sid, F)
    ebase = wid * EPW
    plsc.subcore_barrier()

    def _pair(g, _):
        c0 = 2 * g
        base = pl.multiple_of(ebase + c0 * CH, CH)
        pltpu.sync_copy(src_hbm.at[pl.ds(base, CH)], idxs0)
        pltpu.sync_copy(src_hbm.at[pl.ds(base + CH, CH)], idxs1)
        d0 = pltpu.async_copy(y_hbm.at[idxs0], rows0, gs0)
        d1 = pltpu.async_copy(y_hbm.at[idxs1], rows1, gs1)
        pltpu.sync_copy(dst_hbm.at[pl.ds(base, CH)], idxd0)
        pltpu.sync_copy(dst_hbm.at[pl.ds(base + CH, CH)], idxd1)
        d0.wait()
        s0 = pltpu.async_copy(rows0, acc.at[idxd0], ss0, add=True)
        d1.wait()
        s1 = pltpu.async_copy(rows1, acc.at[idxd1], ss1, add=True)
        s0.wait()
        s1.wait()
        return 0

    lax.fori_loop(0, NCHUNK // 2, _pair, 0)

    plsc.subcore_barrier()
    pltpu.sync_copy(acc.at[pl.ds(sid * RPT, RPT)],
                    out_hbm.at[cid, pl.ds(sid * RPT, RPT)])


_ecc_call = functools.partial(
    pl.kernel,
    _ecc_body,
    out_type=jax.ShapeDtypeStruct((NC, NPAD, 2 * C1), jnp.float32),
    mesh=_mesh,
    scratch_types=[
        pltpu.VMEM((ECH,), jnp.int32),
        pltpu.VMEM((ECH,), jnp.int32),
        pltpu.VMEM((ECH, TGW), jnp.float32),
        pltpu.VMEM((HROWS, F), jnp.float32),
        pltpu.VMEM((ECH, 2 * C1), jnp.float32),
        pltpu.VMEM((8, 2 * C1), jnp.float32),
        pltpu.VMEM_SHARED((NPAD, 2 * C1), jnp.float32),
        pltpu.SemaphoreType.DMA,
    ],
)


_seg_call = functools.partial(
    pl.kernel,
    _seg_body,
    out_type=jax.ShapeDtypeStruct((NC, NPAD, F), jnp.float32),
    mesh=_mesh,
    scratch_types=[
        pltpu.VMEM((CH,), jnp.int32),
        pltpu.VMEM((CH,), jnp.int32),
        pltpu.VMEM((CH,), jnp.int32),
        pltpu.VMEM((CH,), jnp.int32),
        pltpu.VMEM((CH, F), jnp.float32),
        pltpu.VMEM((CH, F), jnp.float32),
        pltpu.VMEM((8, F), jnp.float32),
        pltpu.VMEM_SHARED((NPAD, F), jnp.float32),
        pltpu.SemaphoreType.DMA,
        pltpu.SemaphoreType.DMA,
        pltpu.SemaphoreType.DMA,
        pltpu.SemaphoreType.DMA,
    ],
)


def kernel(x, edge_index, e, i, W0, b0, W1, b1, W2, b2, Wr, br,
           Wg1, bg1, Wg2, bg2, Wd1, bd1, Wd2, bd2):
    f32 = jnp.float32
    src = edge_index[0].astype(jnp.int32)
    dst = edge_index[1].astype(jnp.int32)
    pad_idx = jnp.full((EPAD - E,), N, jnp.int32)
    src_p = jnp.concatenate([src, pad_idx])
    dst_p = jnp.concatenate([dst, pad_idx])
    src2 = src_p.reshape(EPAD // CH, CH)
    dst2 = dst_p.reshape(EPAD // CH, CH)
    e_p = jnp.pad(e, ((0, EPAD - E), (0, 0)))

    W2r = W2.reshape(H, F, C1)
    wcat = jnp.concatenate(
        [jnp.transpose(W2r, (1, 0, 2)).reshape(F, H * C1), b2.reshape(F, C1),
         jnp.zeros((F, TGW - H * C1 - C1), f32)],
        axis=1)
    W1p = jnp.pad(W1, ((0, 0), (0, C2 - H)))
    b1p = jnp.pad(b1, (0, C2 - H)).reshape(1, C2)
    b0r = b0.reshape(1, H)
    brr = br.reshape(1, C1)
    bg1r = bg1.reshape(1, C2)
    bg2r = bg2.reshape(1, C2)
    bd1r = bd1.reshape(1, 16)
    bd2r = bd2.reshape(1, 1)
    i_p = jnp.pad(i.astype(f32), (0, NPAD - N),
                  constant_values=-1.0).reshape(NPAD // BM, 1, BM)

    tg = pl.pallas_call(
        _mm_body,
        grid=(NPAD // BM,),
        in_specs=[pl.BlockSpec((BM, F), lambda t: (t, 0)),
                  pl.BlockSpec((F, TGW), lambda t: (0, 0))],
        out_specs=pl.BlockSpec((BM, TGW), lambda t: (t, 0)),
        out_shape=jax.ShapeDtypeStruct((NPAD, TGW), f32),
    )(x, wcat)

    BE = 2048
    h_e = pl.pallas_call(
        _emlp_body,
        grid=(EPAD // BE,),
        in_specs=[pl.BlockSpec((BE, DE), lambda t: (t, 0)),
                  pl.BlockSpec((DE, H), lambda t: (0, 0)),
                  pl.BlockSpec((1, H), lambda t: (0, 0)),
                  pl.BlockSpec((H, C2), lambda t: (0, 0)),
                  pl.BlockSpec((1, C2), lambda t: (0, 0))],
        out_specs=pl.BlockSpec((BE, C2), lambda t: (t, 0)),
        out_shape=jax.ShapeDtypeStruct((EPAD, C2), f32),
    )(e_p, W0, b0r, W1p, b1p)

    _USE_SC = {"ecc": True, "seg1": True, "seg2": True}
    if _USE_SC["ecc"]:
        h4 = h_e.reshape(EPAD // 4, F)
        agg2 = _ecc_call()(tg, h4, src_p, dst_p)
    else:
        rowsg = tg[src_p]
        msum = ((rowsg[:, :H * C1].reshape(EPAD, H, C1)
                 * h_e[:, :H, None]).sum(1) + rowsg[:, H * C1:H * C1 + C1])
        wide = jnp.concatenate(
            [msum, jnp.ones((EPAD, 1), f32), jnp.zeros((EPAD, 15), f32)], axis=1)
        a0 = jax.ops.segment_sum(wide, dst_p, num_segments=NPAD)
        agg2 = jnp.stack([a0, jnp.zeros_like(a0)])

    y1, n32 = pl.pallas_call(
        _node1_body,
        grid=(NPAD // BM,),
        in_specs=[pl.BlockSpec((BM, 2 * C1), lambda t: (t, 0)),
                  pl.BlockSpec((BM, 2 * C1), lambda t: (t, 0)),
                  pl.BlockSpec((BM, F), lambda t: (t, 0)),
                  pl.BlockSpec((F, C1), lambda t: (0, 0)),
                  pl.BlockSpec((1, C1), lambda t: (0, 0))],
        out_specs=[pl.BlockSpec((BM, F), lambda t: (t, 0)),
                   pl.BlockSpec((BM, C2), lambda t: (t, 0))],
        out_shape=[jax.ShapeDtypeStruct((NPAD, F), f32),
                   jax.ShapeDtypeStruct((NPAD, C2), f32)],
    )(agg2[0], agg2[1], x, Wr, brr)

    if _USE_SC["seg1"]:
        s1 = _seg_call()(y1, src_p, dst_p)
    else:
        s1a = jax.ops.segment_sum(y1[src_p], dst_p, num_segments=NPAD)
        s1 = jnp.stack([s1a, jnp.zeros_like(s1a)])

    y2 = pl.pallas_call(
        _node2_body,
        grid=(NPAD // BM,),
        in_specs=[pl.BlockSpec((BM, F), lambda t: (t, 0)),
                  pl.BlockSpec((BM, F), lambda t: (t, 0)),
                  pl.BlockSpec((BM, F), lambda t: (t, 0)),
                  pl.BlockSpec((BM, C2), lambda t: (t, 0)),
                  pl.BlockSpec((C1, C2), lambda t: (0, 0)),
                  pl.BlockSpec((1, C2), lambda t: (0, 0))],
        out_specs=pl.BlockSpec((BM, F), lambda t: (t, 0)),
        out_shape=jax.ShapeDtypeStruct((NPAD, F), f32),
    )(s1[0], s1[1], y1, n32, Wg1, bg1r)

    if _USE_SC["seg2"]:
        s2 = _seg_call()(y2, src_p, dst_p)
    else:
        s2a = jax.ops.segment_sum(y2[src_p], dst_p, num_segments=NPAD)
        s2 = jnp.stack([s2a, jnp.zeros_like(s2a)])

    out = pl.pallas_call(
        _final_body,
        grid=(NPAD // BM,),
        in_specs=[pl.BlockSpec((BM, F), lambda t: (t, 0)),
                  pl.BlockSpec((BM, F), lambda t: (t, 0)),
                  pl.BlockSpec((BM, F), lambda t: (t, 0)),
                  pl.BlockSpec((BM, C2), lambda t: (t, 0)),
                  pl.BlockSpec((1, 1, BM), lambda t: (t, 0, 0)),
                  pl.BlockSpec((C2, C2), lambda t: (0, 0)),
                  pl.BlockSpec((1, C2), lambda t: (0, 0)),
                  pl.BlockSpec((C2, 16), lambda t: (0, 0)),
                  pl.BlockSpec((1, 16), lambda t: (0, 0)),
                  pl.BlockSpec((16, 1), lambda t: (0, 0)),
                  pl.BlockSpec((1, 1), lambda t: (0, 0))],
        out_specs=pl.BlockSpec((G, 1), lambda t: (0, 0)),
        out_shape=jax.ShapeDtypeStruct((G, 1), f32),
        scratch_shapes=[pltpu.VMEM((G, C2), f32)],
    )(s2[0], s2[1], y2, n32, i_p, Wg2, bg2r, Wd1, bd1r, Wd2, bd2r)

    return out


# speedup vs baseline: 3.0775x; 1.0081x over previous
import functools

import jax
import jax.numpy as jnp
from jax import lax
from jax.experimental import pallas as pl
from jax.experimental.pallas import tpu as pltpu
from jax.experimental.pallas import tpu_sc as plsc

N = 10000
E = 160000
F = 128
DE = 4
H = 20
C1 = 16
C2 = 32
G = 64

NC = 2
NS = 16
NW = NC * NS
CH = 128
NPAD = 10112
EPAD = 163840
EPW = EPAD // NW
NCHUNK = EPW // CH
ECH = 64
ECHUNK = EPW // ECH
NSLOT = 3
NTRI = ECHUNK // NSLOT
HROWS = ECH // 4
RPT = NPAD // NS
TGW = 384
BM = 2528

_mesh = plsc.VectorSubcoreMesh(core_axis_name="c", subcore_axis_name="s")


def _mm_body(x_ref, w_ref, o_ref):
    o_ref[...] = x_ref[...] @ w_ref[...]


def _emlp_body(e_ref, w0_ref, b0_ref, w1_ref, b1_ref, o_ref):
    h1 = jnp.maximum(e_ref[...] @ w0_ref[...] + b0_ref[...], 0.0)
    o_ref[...] = jnp.maximum(h1 @ w1_ref[...] + b1_ref[...], 0.0)


def _node1_body(a0_ref, a1_ref, x_ref, wr_ref, br_ref, y1_ref, n32_ref):
    a0 = a0_ref[...]
    a1 = a1_ref[...]
    agg = a0[:, :C1] + a1[:, :C1]
    deg = a0[:, C1:C1 + 1] + a1[:, C1:C1 + 1] + 1.0
    x1 = jnp.maximum(agg + x_ref[...] @ wr_ref[...] + br_ref[...], 0.0)
    nrm = lax.rsqrt(deg)
    y1 = x1 * nrm
    y1_ref[...] = jnp.concatenate([y1] * (F // C1), axis=1)
    n32_ref[...] = jnp.broadcast_to(nrm, n32_ref.shape)


def _node2_body(sa_ref, sb_ref, y1_ref, n32_ref, wg_ref, bg_ref, y2_ref):
    n32 = n32_ref[...]
    p = n32[:, :C1] * (sa_ref[:, :C1] + sb_ref[:, :C1] + y1_ref[:, :C1])
    x2 = jnp.maximum(p @ wg_ref[...] + bg_ref[...], 0.0)
    y2_ref[...] = jnp.concatenate([x2 * n32] * (F // C2), axis=1)


def _final_body(sa_ref, sb_ref, y2_ref, n32_ref, irow_ref, wg_ref, bg_ref,
                wd1_ref, bd1_ref, wd2_ref, bd2_ref, o_ref, pool_ref):
    t = pl.program_id(0)
    p = n32_ref[...] * (sa_ref[:, :C2] + sb_ref[:, :C2] + y2_ref[:, :C2])
    x3 = jnp.maximum(p @ wg_ref[...] + bg_ref[...], 0.0)
    rid = t * BM + lax.broadcasted_iota(jnp.int32, (BM, 1), 0)
    x3 = jnp.where(rid < N, x3, 0.0)
    gid = irow_ref[0]
    oh = (lax.broadcasted_iota(jnp.int32, (G, BM), 0).astype(jnp.float32)
          == gid)
    contrib = oh.astype(jnp.float32) @ x3

    @pl.when(t == 0)
    def _():
        pool_ref[...] = jnp.zeros_like(pool_ref)

    pool_ref[...] += contrib

    @pl.when(t == pl.num_programs(0) - 1)
    def _():
        g = pool_ref[...]
        o_ref[...] = (g @ wd1_ref[...] + bd1_ref[...]) @ wd2_ref[...] + bd2_ref[...]


def _zero_shared(zbuf, acc, sid, width):
    nv = width // 16
    zr = 64

    def _z(j, _):
        for v in range(nv):
            zbuf[j, pl.ds(v * 16, 16)] = jnp.zeros((16,), jnp.float32)
        return 0

    lax.fori_loop(0, zr, _z, 0)

    def _cp(q, _):
        off = pl.multiple_of(sid * RPT + q * zr, 8)
        pltpu.sync_copy(zbuf, acc.at[pl.ds(off, zr)])
        return 0

    lax.fori_loop(0, RPT // zr, _cp, 0)
    tail = RPT - (RPT // zr) * zr
    off = pl.multiple_of(sid * RPT + (RPT // zr) * zr, 8)
    pltpu.sync_copy(zbuf.at[pl.ds(0, tail)], acc.at[pl.ds(off, tail)])


def _ecc_compute(rows, hbuf, msg):
    def _edge(k, _):
        r = k // 4
        off = (k - 4 * r) * 32
        hv0 = hbuf[r, pl.ds(off, 16)]
        hv1 = hbuf[r, pl.ds(off + 16, 16)]
        m = rows[k, pl.ds(H * C1, 16)]
        for hh in range(H):
            s = hv0[hh] if hh < 16 else hv1[hh - 16]
            m = m + rows[k, pl.ds(hh * 16, 16)] * s
        msg[k, pl.ds(0, 16)] = m
        return 0

    lax.fori_loop(0, ECH, _edge, 0)


def _ecc_body(tg_hbm, h_hbm, src_hbm, dst_hbm, out_hbm,
              idx_s, idx_d, rows, hbuf, msg, zbuf, acc, sem):
    cid = lax.axis_index("c")
    sid = lax.axis_index("s")
    wid = sid * NC + cid

    _zero_shared(zbuf, acc, sid, 2 * C1)

    cvec = jnp.where(lax.iota(jnp.int32, 16) == 0, 1.0, 0.0)

    def _c(j, _):
        msg[j, pl.ds(C1, 16)] = cvec
        return 0

    lax.fori_loop(0, ECH, _c, 0)

    ebase = wid * EPW
    hbase0 = wid * (EPW // 4)
    plsc.subcore_barrier()

    def _chunk(c, _):
        base = pl.multiple_of(ebase + c * ECH, ECH)
        hbase = pl.multiple_of(hbase0 + c * HROWS, 8)
        pltpu.sync_copy(src_hbm.at[pl.ds(base, ECH)], idx_s)
        pltpu.sync_copy(dst_hbm.at[pl.ds(base, ECH)], idx_d)
        pltpu.sync_copy(h_hbm.at[pl.ds(hbase, HROWS)], hbuf)
        pltpu.async_copy(tg_hbm.at[idx_s], rows, sem).wait()
        _ecc_compute(rows, hbuf, msg)
        pltpu.sync_copy(msg, acc.at[idx_d], add=True)
        return 0

    lax.fori_loop(0, ECHUNK, _chunk, 0)

    plsc.subcore_barrier()
    pltpu.sync_copy(acc.at[pl.ds(sid * RPT, RPT)],
                    out_hbm.at[cid, pl.ds(sid * RPT, RPT)])


def _seg_body(y_hbm, src_hbm, dst_hbm, out_hbm,
              idxs0, idxs1, idxd0, idxd1, rows0, rows1, zbuf, acc,
              gs0, gs1, ss0, ss1):
    cid = lax.axis_index("c")
    sid = lax.axis_index("s")
    wid = sid * NC + cid

    _zero_shared(zbuf, acc, sid, F)
    ebase = wid * EPW
    plsc.subcore_barrier()

    def _pair(g, _):
        c0 = 2 * g
        base = pl.multiple_of(ebase + c0 * CH, CH)
        pltpu.sync_copy(src_hbm.at[pl.ds(base, CH)], idxs0)
        pltpu.sync_copy(src_hbm.at[pl.ds(base + CH, CH)], idxs1)
        d0 = pltpu.async_copy(y_hbm.at[idxs0], rows0, gs0)
        d1 = pltpu.async_copy(y_hbm.at[idxs1], rows1, gs1)
        pltpu.sync_copy(dst_hbm.at[pl.ds(base, CH)], idxd0)
        pltpu.sync_copy(dst_hbm.at[pl.ds(base + CH, CH)], idxd1)
        d0.wait()
        s0 = pltpu.async_copy(rows0, acc.at[idxd0], ss0, add=True)
        d1.wait()
        s1 = pltpu.async_copy(rows1, acc.at[idxd1], ss1, add=True)
        s0.wait()
        s1.wait()
        return 0

    lax.fori_loop(0, NCHUNK // 2, _pair, 0)

    plsc.subcore_barrier()
    pltpu.sync_copy(acc.at[pl.ds(sid * RPT, RPT)],
                    out_hbm.at[cid, pl.ds(sid * RPT, RPT)])


_ecc_call = functools.partial(
    pl.kernel,
    _ecc_body,
    out_type=jax.ShapeDtypeStruct((NC, NPAD, 2 * C1), jnp.float32),
    mesh=_mesh,
    scratch_types=[
        pltpu.VMEM((ECH,), jnp.int32),
        pltpu.VMEM((ECH,), jnp.int32),
        pltpu.VMEM((ECH, TGW), jnp.float32),
        pltpu.VMEM((HROWS, F), jnp.float32),
        pltpu.VMEM((ECH, 2 * C1), jnp.float32),
        pltpu.VMEM((64, 2 * C1), jnp.float32),
        pltpu.VMEM_SHARED((NPAD, 2 * C1), jnp.float32),
        pltpu.SemaphoreType.DMA,
    ],
)


_seg_call = functools.partial(
    pl.kernel,
    _seg_body,
    out_type=jax.ShapeDtypeStruct((NC, NPAD, F), jnp.float32),
    mesh=_mesh,
    scratch_types=[
        pltpu.VMEM((CH,), jnp.int32),
        pltpu.VMEM((CH,), jnp.int32),
        pltpu.VMEM((CH,), jnp.int32),
        pltpu.VMEM((CH,), jnp.int32),
        pltpu.VMEM((CH, F), jnp.float32),
        pltpu.VMEM((CH, F), jnp.float32),
        pltpu.VMEM((64, F), jnp.float32),
        pltpu.VMEM_SHARED((NPAD, F), jnp.float32),
        pltpu.SemaphoreType.DMA,
        pltpu.SemaphoreType.DMA,
        pltpu.SemaphoreType.DMA,
        pltpu.SemaphoreType.DMA,
    ],
)


def kernel(x, edge_index, e, i, W0, b0, W1, b1, W2, b2, Wr, br,
           Wg1, bg1, Wg2, bg2, Wd1, bd1, Wd2, bd2):
    f32 = jnp.float32
    src = edge_index[0].astype(jnp.int32)
    dst = edge_index[1].astype(jnp.int32)
    pad_idx = jnp.full((EPAD - E,), N, jnp.int32)
    src_p = jnp.concatenate([src, pad_idx])
    dst_p = jnp.concatenate([dst, pad_idx])
    src2 = src_p.reshape(EPAD // CH, CH)
    dst2 = dst_p.reshape(EPAD // CH, CH)
    e_p = jnp.pad(e, ((0, EPAD - E), (0, 0)))

    W2r = W2.reshape(H, F, C1)
    wcat = jnp.concatenate(
        [jnp.transpose(W2r, (1, 0, 2)).reshape(F, H * C1), b2.reshape(F, C1),
         jnp.zeros((F, TGW - H * C1 - C1), f32)],
        axis=1)
    W1p = jnp.pad(W1, ((0, 0), (0, C2 - H)))
    b1p = jnp.pad(b1, (0, C2 - H)).reshape(1, C2)
    b0r = b0.reshape(1, H)
    brr = br.reshape(1, C1)
    bg1r = bg1.reshape(1, C2)
    bg2r = bg2.reshape(1, C2)
    bd1r = bd1.reshape(1, 16)
    bd2r = bd2.reshape(1, 1)
    i_p = jnp.pad(i.astype(f32), (0, NPAD - N),
                  constant_values=-1.0).reshape(NPAD // BM, 1, BM)

    tg = pl.pallas_call(
        _mm_body,
        grid=(NPAD // BM,),
        in_specs=[pl.BlockSpec((BM, F), lambda t: (t, 0)),
                  pl.BlockSpec((F, TGW), lambda t: (0, 0))],
        out_specs=pl.BlockSpec((BM, TGW), lambda t: (t, 0)),
        out_shape=jax.ShapeDtypeStruct((NPAD, TGW), f32),
    )(x, wcat)

    BE = 2048
    h_e = pl.pallas_call(
        _emlp_body,
        grid=(EPAD // BE,),
        in_specs=[pl.BlockSpec((BE, DE), lambda t: (t, 0)),
                  pl.BlockSpec((DE, H), lambda t: (0, 0)),
                  pl.BlockSpec((1, H), lambda t: (0, 0)),
                  pl.BlockSpec((H, C2), lambda t: (0, 0)),
                  pl.BlockSpec((1, C2), lambda t: (0, 0))],
        out_specs=pl.BlockSpec((BE, C2), lambda t: (t, 0)),
        out_shape=jax.ShapeDtypeStruct((EPAD, C2), f32),
    )(e_p, W0, b0r, W1p, b1p)

    _USE_SC = {"ecc": True, "seg1": True, "seg2": True}
    if _USE_SC["ecc"]:
        h4 = h_e.reshape(EPAD // 4, F)
        agg2 = _ecc_call()(tg, h4, src_p, dst_p)
    else:
        rowsg = tg[src_p]
        msum = ((rowsg[:, :H * C1].reshape(EPAD, H, C1)
                 * h_e[:, :H, None]).sum(1) + rowsg[:, H * C1:H * C1 + C1])
        wide = jnp.concatenate(
            [msum, jnp.ones((EPAD, 1), f32), jnp.zeros((EPAD, 15), f32)], axis=1)
        a0 = jax.ops.segment_sum(wide, dst_p, num_segments=NPAD)
        agg2 = jnp.stack([a0, jnp.zeros_like(a0)])

    y1, n32 = pl.pallas_call(
        _node1_body,
        grid=(NPAD // BM,),
        in_specs=[pl.BlockSpec((BM, 2 * C1), lambda t: (t, 0)),
                  pl.BlockSpec((BM, 2 * C1), lambda t: (t, 0)),
                  pl.BlockSpec((BM, F), lambda t: (t, 0)),
                  pl.BlockSpec((F, C1), lambda t: (0, 0)),
                  pl.BlockSpec((1, C1), lambda t: (0, 0))],
        out_specs=[pl.BlockSpec((BM, F), lambda t: (t, 0)),
                   pl.BlockSpec((BM, C2), lambda t: (t, 0))],
        out_shape=[jax.ShapeDtypeStruct((NPAD, F), f32),
                   jax.ShapeDtypeStruct((NPAD, C2), f32)],
    )(agg2[0], agg2[1], x, Wr, brr)

    if _USE_SC["seg1"]:
        s1 = _seg_call()(y1, src_p, dst_p)
    else:
        s1a = jax.ops.segment_sum(y1[src_p], dst_p, num_segments=NPAD)
        s1 = jnp.stack([s1a, jnp.zeros_like(s1a)])

    y2 = pl.pallas_call(
        _node2_body,
        grid=(NPAD // BM,),
        in_specs=[pl.BlockSpec((BM, F), lambda t: (t, 0)),
                  pl.BlockSpec((BM, F), lambda t: (t, 0)),
                  pl.BlockSpec((BM, F), lambda t: (t, 0)),
                  pl.BlockSpec((BM, C2), lambda t: (t, 0)),
                  pl.BlockSpec((C1, C2), lambda t: (0, 0)),
                  pl.BlockSpec((1, C2), lambda t: (0, 0))],
        out_specs=pl.BlockSpec((BM, F), lambda t: (t, 0)),
        out_shape=jax.ShapeDtypeStruct((NPAD, F), f32),
    )(s1[0], s1[1], y1, n32, Wg1, bg1r)

    if _USE_SC["seg2"]:
        s2 = _seg_call()(y2, src_p, dst_p)
    else:
        s2a = jax.ops.segment_sum(y2[src_p], dst_p, num_segments=NPAD)
        s2 = jnp.stack([s2a, jnp.zeros_like(s2a)])

    out = pl.pallas_call(
        _final_body,
        grid=(NPAD // BM,),
        in_specs=[pl.BlockSpec((BM, F), lambda t: (t, 0)),
                  pl.BlockSpec((BM, F), lambda t: (t, 0)),
                  pl.BlockSpec((BM, F), lambda t: (t, 0)),
                  pl.BlockSpec((BM, C2), lambda t: (t, 0)),
                  pl.BlockSpec((1, 1, BM), lambda t: (t, 0, 0)),
                  pl.BlockSpec((C2, C2), lambda t: (0, 0)),
                  pl.BlockSpec((1, C2), lambda t: (0, 0)),
                  pl.BlockSpec((C2, 16), lambda t: (0, 0)),
                  pl.BlockSpec((1, 16), lambda t: (0, 0)),
                  pl.BlockSpec((16, 1), lambda t: (0, 0)),
                  pl.BlockSpec((1, 1), lambda t: (0, 0))],
        out_specs=pl.BlockSpec((G, 1), lambda t: (0, 0)),
        out_shape=jax.ShapeDtypeStruct((G, 1), f32),
        scratch_shapes=[pltpu.VMEM((G, C2), f32)],
    )(s2[0], s2[1], y2, n32, i_p, Wg2, bg2r, Wd1, bd1r, Wd2, bd2r)

    return out
